# baseline (speedup 1.0000x reference)
import functools

import jax
import jax.numpy as jnp
from jax import lax
from jax.experimental import pallas as pl
from jax.experimental.pallas import tpu as pltpu
from jax.experimental.pallas import tpu_sc as plsc

_NUM_LABELS = 20
_NUM_POS = 8
_NUM_NEG = 16
_POS_TH = 0.2
_NEG_TH = 1.0
_NEG_W = 1.0
_IGNORE = 255
_EPS = 1e-4
_N = 100000
_D = 64

_NC = 2
_NS = 16
_NW = _NC * _NS

_LT = 32

_PB = 80
_PCHUNKS = _N // _PB

_B = 32
_S = _NUM_POS + _NUM_NEG
_PAIRS = _B * _S
_ISTREAM = 128
_NSTREAM = _PAIRS // _ISTREAM
_CHUNKS = _N // _B

_mesh = plsc.VectorSubcoreMesh(core_axis_name="c", subcore_axis_name="s")
_cparams = pltpu.CompilerParams(use_tc_tiling_on_sc=False,
                                needs_layout_passes=False)


def _permute_body(feats_hbm, order_hbm, sorted_hbm, idx_v, rows_v, sem, gsem):
    wid = lax.axis_index("s") * _NC + lax.axis_index("c")
    nloops = (_PCHUNKS - wid + _NW - 1) // _NW

    def chunk(i, carry):
        base = (wid + i * _NW) * _PB
        pltpu.sync_copy(order_hbm.at[pl.ds(base, _PB)], idx_v)
        cp = pltpu.make_async_copy(feats_hbm.at[idx_v], rows_v, gsem)
        cp.start()
        cp.wait()
        pltpu.sync_copy(rows_v, sorted_hbm.at[pl.ds(base, _PB)])
        return carry

    lax.fori_loop(0, nloops, chunk, 0)


_permute = functools.partial(
    pl.kernel,
    out_type=jax.ShapeDtypeStruct((_N, _D), jnp.float32),
    mesh=_mesh,
    scratch_types=[
        pltpu.VMEM((_PB,), jnp.int32),
        pltpu.VMEM((_PB, _D), jnp.float32),
        pltpu.SemaphoreType.DMA,
        pltpu.SemaphoreType.DMA,
    ],
    compiler_params=_cparams,
)(_permute_body)


def _rsqrt_newton(y):
    yi = plsc.bitcast(y, jnp.int32)
    h = plsc.bitcast(jnp.int32(0x5F3759DF) - (yi >> 1), jnp.float32)
    for _ in range(3):
        h = h * (1.5 - 0.5 * y * h * h)
    return h


def _main_body(sorted_hbm, feats_hbm, labels_hbm, u_hbm, u2_hbm, negc_hbm,
               counts_hbm, offsets_hbm, pos_hbm, neg_hbm,
               cnt_v, off_v, lab_v, u_v, u2_v, negc_v, f_v, idx_v, rows_v,
               posl_v, negl_v, isem, gsem):
    wid = lax.axis_index("s") * _NC + lax.axis_index("c")
    pltpu.sync_copy(counts_hbm, cnt_v[0])
    pltpu.sync_copy(offsets_hbm, off_v[0])
    lane = lax.iota(jnp.int32, 16)
    nloops = (_CHUNKS - wid + _NW - 1) // _NW

    def in_copies(i, s):
        base = (wid + i * _NW) * _B
        return [
            pltpu.make_async_copy(labels_hbm.at[pl.ds(base, _B)], lab_v[s], isem[s]),
            pltpu.make_async_copy(u_hbm.at[:, pl.ds(base, _B)], u_v[s], isem[s]),
            pltpu.make_async_copy(u2_hbm.at[:, pl.ds(base, _B)], u2_v[s], isem[s]),
            pltpu.make_async_copy(negc_hbm.at[:, pl.ds(base, _B)], negc_v[s], isem[s]),
            pltpu.make_async_copy(feats_hbm.at[pl.ds(base, _B)], f_v[s], isem[s]),
        ]

    def start_inputs(i, s):
        for cp in in_copies(i, s):
            cp.start()

    def wait_inputs(i, s):
        for cp in in_copies(i, s):
            cp.wait()

    def sample_and_start_gather(i, s):
        for g in range(_B // 16):
            b0 = g * 16
            l = lab_v[s][pl.ds(b0, 16)]
            cnt = plsc.load_gather(cnt_v[0], [l])
            off = plsc.load_gather(off_v[0], [l])
            cntf = cnt.astype(jnp.float32)
            cm1 = cnt - 1
            pb = (lane + b0) * _S
            for p in range(_NUM_POS):
                u = u_v[s][p, pl.ds(b0, 16)]
                r = jnp.minimum((u * cntf).astype(jnp.int32), cm1)
                plsc.store_scatter(idx_v[s], [pb + p], off + r)
            for k in range(_NUM_NEG):
                c = negc_v[s][k, pl.ds(b0, 16)]
                cc = plsc.load_gather(cnt_v[0], [c])
                offc = plsc.load_gather(off_v[0], [c])
                u2 = u2_v[s][k, pl.ds(b0, 16)]
                r = jnp.minimum((u2 * cc.astype(jnp.float32)).astype(jnp.int32),
                                jnp.maximum(cc - 1, 0))
                j = jnp.minimum(offc + r, _N - 1)
                plsc.store_scatter(idx_v[s], [pb + _NUM_POS + k], j)
        pltpu.make_async_copy(sorted_hbm.at[idx_v[s]], rows_v[s], gsem[s]).start()

    def finish(i, s):
        pltpu.make_async_copy(sorted_hbm.at[idx_v[s]], rows_v[s], gsem[s]).wait()
        base = (wid + i * _NW) * _B
        for g in range(_B // 16):
            b0 = g * 16
            l = lab_v[s][pl.ds(b0, 16)]
            pb = (lane + b0) * _S
            bvec = lane + b0

            def sblock(s0):
                def dstep(di, accs):
                    out = list(accs)
                    for dd in range(4):
                        dsp = jnp.full((16,), di * 4 + dd, jnp.int32)
                        fd = plsc.load_gather(f_v[s], [bvec, dsp])
                        for t in range(8):
                            rv = plsc.load_gather(rows_v[s], [pb + s0 + t, dsp])
                            df = fd - rv
                            out[t] = out[t] + df * df
                    return tuple(out)

                return lax.fori_loop(
                    0, _D // 4, dstep,
                    tuple(jnp.zeros((16,), jnp.float32) for _ in range(8)))

            accs = sblock(0) + sblock(8) + sblock(16)

            dists = []
            for t in range(_S):
                y = accs[t] + 1e-7
                dists.append(y * _rsqrt_newton(y))
            dpos = dists[0]
            for t in range(1, _NUM_POS):
                dpos = dpos + dists[t]
            dpos = dpos * (1.0 / _NUM_POS)
            dneg = dists[_NUM_POS]
            for t in range(_NUM_POS + 1, _S):
                dneg = dneg + dists[t]
            dneg = dneg * (1.0 / _NUM_NEG)

            valid = l != _IGNORE
            posl = jnp.where(valid, jnp.maximum(dpos - _POS_TH, 0.0), 0.0)
            negl = jnp.where(valid, jnp.maximum(_NEG_TH - dneg, 0.0), 0.0)
            posl_v[s][pl.ds(b0, 16)] = posl
            negl_v[s][pl.ds(b0, 16)] = negl

        pltpu.sync_copy(posl_v[s], pos_hbm.at[pl.ds(base, _B)])
        pltpu.sync_copy(negl_v[s], neg_hbm.at[pl.ds(base, _B)])

    def iteration(i, s):
        @pl.when(i < nloops)
        def _():
            wait_inputs(i, s)
            sample_and_start_gather(i, s)

        @pl.when(jnp.logical_and(i >= 1, i - 1 < nloops))
        def _():
            finish(i - 1, 1 - s)

        @pl.when(i + 1 < nloops)
        def _():
            start_inputs(i + 1, 1 - s)

    start_inputs(0, 0)

    def pair(k, carry):
        iteration(2 * k, 0)
        iteration(2 * k + 1, 1)
        return carry

    lax.fori_loop(0, (nloops + 2) // 2, pair, 0)


_main = functools.partial(
    pl.kernel,
    out_type=(jax.ShapeDtypeStruct((_N,), jnp.float32),
              jax.ShapeDtypeStruct((_N,), jnp.float32)),
    mesh=_mesh,
    scratch_types=[
        [pltpu.VMEM((_LT,), jnp.int32)],
        [pltpu.VMEM((_LT,), jnp.int32)],
        [pltpu.VMEM((_B,), jnp.int32)] * 2,
        [pltpu.VMEM((_NUM_POS, _B), jnp.float32)] * 2,
        [pltpu.VMEM((_NUM_NEG, _B), jnp.float32)] * 2,
        [pltpu.VMEM((_NUM_NEG, _B), jnp.int32)] * 2,
        [pltpu.VMEM((_B, _D), jnp.float32)] * 2,
        [pltpu.VMEM((_PAIRS,), jnp.int32)] * 2,
        [pltpu.VMEM((_PAIRS, _D), jnp.float32)] * 2,
        [pltpu.VMEM((_B,), jnp.float32)] * 2,
        [pltpu.VMEM((_B,), jnp.float32)] * 2,
        [pltpu.SemaphoreType.DMA] * 2,
        [pltpu.SemaphoreType.DMA] * 2,
    ],
    compiler_params=_cparams,
)(_main_body)


@jax.jit
def kernel(features, labels, confusion_hist):
    labels = labels.astype(jnp.int32)
    counts = jnp.bincount(labels, length=_NUM_LABELS)
    order = jnp.argsort(labels).astype(jnp.int32)
    offsets = jnp.cumsum(counts) - counts

    key = jax.random.key(42)
    kp, kc, ku = jax.random.split(key, 3)
    u = jax.random.uniform(kp, (_N, _NUM_POS))
    offdiag = 1.0 - jnp.eye(_NUM_LABELS, dtype=jnp.float32)
    present = (counts > 0).astype(jnp.float32)
    W = confusion_hist.astype(jnp.float32) * offdiag * present[None, :]
    class_w = W * counts[None, :].astype(jnp.float32)
    logits = jnp.log(class_w + _EPS)
    logits_pp = logits[labels]
    g = jax.random.gumbel(kc, (_N, _NUM_NEG, _NUM_LABELS), jnp.float32)
    neg_cls = jnp.argmax(g + logits_pp[:, None, :], axis=-1).astype(jnp.int32)
    u2 = jax.random.uniform(ku, (_N, _NUM_NEG))

    counts_p = jnp.concatenate(
        [counts.astype(jnp.int32), jnp.ones((_LT - _NUM_LABELS,), jnp.int32)])
    offsets_p = jnp.concatenate(
        [offsets.astype(jnp.int32), jnp.zeros((_LT - _NUM_LABELS,), jnp.int32)])

    sorted_feats = _permute(features, order)
    pos_loss, neg_loss = _main(
        sorted_feats, features, labels,
        u.T, u2.T, neg_cls.T,
        counts_p, offsets_p)
    loss = pos_loss.mean() + neg_loss.mean() * _NEG_W
    return (loss, pos_loss, neg_loss)


# speedup vs baseline: 1.0081x; 1.0081x over previous
import functools

import jax
import jax.numpy as jnp
from jax import lax
from jax.experimental import pallas as pl
from jax.experimental.pallas import tpu as pltpu
from jax.experimental.pallas import tpu_sc as plsc

_NUM_LABELS = 20
_NUM_POS = 8
_NUM_NEG = 16
_POS_TH = 0.2
_NEG_TH = 1.0
_NEG_W = 1.0
_IGNORE = 255
_EPS = 1e-4
_N = 100000
_D = 64

_NC = 2
_NS = 16
_NW = _NC * _NS

_LT = 32

_PB = 80
_PCHUNKS = _N // _PB

_B = 32
_S = _NUM_POS + _NUM_NEG
_PAIRS = _B * _S
_ISTREAM = 128
_NSTREAM = _PAIRS // _ISTREAM
_CHUNKS = _N // _B
_T = (_CHUNKS + _NW - 1) // _NW

_mesh = plsc.VectorSubcoreMesh(core_axis_name="c", subcore_axis_name="s")
_cparams = pltpu.CompilerParams(use_tc_tiling_on_sc=False,
                                needs_layout_passes=False)


def _permute_body(feats_hbm, order_hbm, sorted_hbm, idx_v, rows_v, sem, gsem):
    wid = lax.axis_index("s") * _NC + lax.axis_index("c")
    nloops = (_PCHUNKS - wid + _NW - 1) // _NW

    def chunk(i, carry):
        base = (wid + i * _NW) * _PB
        pltpu.sync_copy(order_hbm.at[pl.ds(base, _PB)], idx_v)
        cp = pltpu.make_async_copy(feats_hbm.at[idx_v], rows_v, gsem)
        cp.start()
        cp.wait()
        pltpu.sync_copy(rows_v, sorted_hbm.at[pl.ds(base, _PB)])
        return carry

    lax.fori_loop(0, nloops, chunk, 0)


_permute = functools.partial(
    pl.kernel,
    out_type=jax.ShapeDtypeStruct((_N, _D), jnp.float32),
    mesh=_mesh,
    scratch_types=[
        pltpu.VMEM((_PB,), jnp.int32),
        pltpu.VMEM((_PB, _D), jnp.float32),
        pltpu.SemaphoreType.DMA,
        pltpu.SemaphoreType.DMA,
    ],
    compiler_params=_cparams,
)(_permute_body)


def _rsqrt_newton(y):
    yi = plsc.bitcast(y, jnp.int32)
    h = plsc.bitcast(jnp.int32(0x5F3759DF) - (yi >> 1), jnp.float32)
    for _ in range(3):
        h = h * (1.5 - 0.5 * y * h * h)
    return h


def _main_body(sorted_hbm, feats_hbm, labels_hbm, u_hbm, u2_hbm, negc_hbm,
               counts_hbm, offsets_hbm, pos_hbm, neg_hbm,
               cnt_v, off_v, lab_v, u_v, u2_v, negc_v, f_v, idx_v, rows_v,
               posl_v, negl_v, isem, gsem):
    wid = lax.axis_index("s") * _NC + lax.axis_index("c")
    pltpu.sync_copy(counts_hbm, cnt_v[0])
    pltpu.sync_copy(offsets_hbm, off_v[0])
    lane = lax.iota(jnp.int32, 16)
    nloops = (_CHUNKS - wid + _NW - 1) // _NW

    def cbase(i):
        return (wid + jnp.minimum(i, nloops - 1) * _NW) * _B

    def in_copies(i, s):
        base = cbase(i)
        return [
            pltpu.make_async_copy(labels_hbm.at[pl.ds(base, _B)], lab_v[s], isem[s]),
            pltpu.make_async_copy(u_hbm.at[:, pl.ds(base, _B)], u_v[s], isem[s]),
            pltpu.make_async_copy(u2_hbm.at[:, pl.ds(base, _B)], u2_v[s], isem[s]),
            pltpu.make_async_copy(negc_hbm.at[:, pl.ds(base, _B)], negc_v[s], isem[s]),
            pltpu.make_async_copy(feats_hbm.at[pl.ds(base, _B)], f_v[s], isem[s]),
        ]

    def start_inputs(i, s):
        for cp in in_copies(i, s):
            cp.start()

    def wait_inputs(i, s):
        for cp in in_copies(i, s):
            cp.wait()

    def sample_and_start_gather(i, s):
        for g in range(_B // 16):
            b0 = g * 16
            l = lab_v[s][pl.ds(b0, 16)]
            cnt = plsc.load_gather(cnt_v[0], [l])
            off = plsc.load_gather(off_v[0], [l])
            cntf = cnt.astype(jnp.float32)
            cm1 = cnt - 1
            pb = (lane + b0) * _S
            for p in range(_NUM_POS):
                u = u_v[s][p, pl.ds(b0, 16)]
                r = jnp.minimum((u * cntf).astype(jnp.int32), cm1)
                plsc.store_scatter(idx_v[s], [pb + p], off + r)
            for k in range(_NUM_NEG):
                c = negc_v[s][k, pl.ds(b0, 16)]
                cc = plsc.load_gather(cnt_v[0], [c])
                offc = plsc.load_gather(off_v[0], [c])
                u2 = u2_v[s][k, pl.ds(b0, 16)]
                r = jnp.minimum((u2 * cc.astype(jnp.float32)).astype(jnp.int32),
                                jnp.maximum(cc - 1, 0))
                j = jnp.minimum(offc + r, _N - 1)
                plsc.store_scatter(idx_v[s], [pb + _NUM_POS + k], j)
        pltpu.make_async_copy(sorted_hbm.at[idx_v[s]], rows_v[s], gsem[s]).start()

    def finish(i, s):
        pltpu.make_async_copy(sorted_hbm.at[idx_v[s]], rows_v[s], gsem[s]).wait()
        base = cbase(i)
        for g in range(_B // 16):
            b0 = g * 16
            l = lab_v[s][pl.ds(b0, 16)]
            pb = (lane + b0) * _S
            bvec = lane + b0

            def sblock(s0):
                def dstep(di, accs):
                    out = list(accs)
                    for dd in range(4):
                        dsp = jnp.full((16,), di * 4 + dd, jnp.int32)
                        fd = plsc.load_gather(f_v[s], [bvec, dsp])
                        for t in range(8):
                            rv = plsc.load_gather(rows_v[s], [pb + s0 + t, dsp])
                            df = fd - rv
                            out[t] = out[t] + df * df
                    return tuple(out)

                return lax.fori_loop(
                    0, _D // 4, dstep,
                    tuple(jnp.zeros((16,), jnp.float32) for _ in range(8)))

            accs = sblock(0) + sblock(8) + sblock(16)

            dists = []
            for t in range(_S):
                y = accs[t] + 1e-7
                dists.append(y * _rsqrt_newton(y))
            dpos = dists[0]
            for t in range(1, _NUM_POS):
                dpos = dpos + dists[t]
            dpos = dpos * (1.0 / _NUM_POS)
            dneg = dists[_NUM_POS]
            for t in range(_NUM_POS + 1, _S):
                dneg = dneg + dists[t]
            dneg = dneg * (1.0 / _NUM_NEG)

            valid = l != _IGNORE
            posl = jnp.where(valid, jnp.maximum(dpos - _POS_TH, 0.0), 0.0)
            negl = jnp.where(valid, jnp.maximum(_NEG_TH - dneg, 0.0), 0.0)
            posl_v[s][pl.ds(b0, 16)] = posl
            negl_v[s][pl.ds(b0, 16)] = negl

        pltpu.sync_copy(posl_v[s], pos_hbm.at[pl.ds(base, _B)])
        pltpu.sync_copy(negl_v[s], neg_hbm.at[pl.ds(base, _B)])

    def iteration(i, s, do_finish=True):
        wait_inputs(i, s)
        sample_and_start_gather(i, s)
        if do_finish:
            finish(i - 1, 1 - s)
        start_inputs(i + 1, 1 - s)

    start_inputs(0, 0)
    iteration(0, 0, do_finish=False)
    iteration(1, 1)

    def pair(k, carry):
        iteration(2 * k, 0)
        iteration(2 * k + 1, 1)
        return carry

    lax.fori_loop(1, _T // 2, pair, 0)
    finish(_T - 1, (_T - 1) % 2)
    wait_inputs(_T, _T % 2)


_main = functools.partial(
    pl.kernel,
    out_type=(jax.ShapeDtypeStruct((_N,), jnp.float32),
              jax.ShapeDtypeStruct((_N,), jnp.float32)),
    mesh=_mesh,
    scratch_types=[
        [pltpu.VMEM((_LT,), jnp.int32)],
        [pltpu.VMEM((_LT,), jnp.int32)],
        [pltpu.VMEM((_B,), jnp.int32)] * 2,
        [pltpu.VMEM((_NUM_POS, _B), jnp.float32)] * 2,
        [pltpu.VMEM((_NUM_NEG, _B), jnp.float32)] * 2,
        [pltpu.VMEM((_NUM_NEG, _B), jnp.int32)] * 2,
        [pltpu.VMEM((_B, _D), jnp.float32)] * 2,
        [pltpu.VMEM((_PAIRS,), jnp.int32)] * 2,
        [pltpu.VMEM((_PAIRS, _D), jnp.float32)] * 2,
        [pltpu.VMEM((_B,), jnp.float32)] * 2,
        [pltpu.VMEM((_B,), jnp.float32)] * 2,
        [pltpu.SemaphoreType.DMA] * 2,
        [pltpu.SemaphoreType.DMA] * 2,
    ],
    compiler_params=_cparams,
)(_main_body)


@jax.jit
def kernel(features, labels, confusion_hist):
    labels = labels.astype(jnp.int32)
    counts = jnp.bincount(labels, length=_NUM_LABELS)
    order = jnp.argsort(labels).astype(jnp.int32)
    offsets = jnp.cumsum(counts) - counts

    key = jax.random.key(42)
    kp, kc, ku = jax.random.split(key, 3)
    u = jax.random.uniform(kp, (_N, _NUM_POS))
    offdiag = 1.0 - jnp.eye(_NUM_LABELS, dtype=jnp.float32)
    present = (counts > 0).astype(jnp.float32)
    W = confusion_hist.astype(jnp.float32) * offdiag * present[None, :]
    class_w = W * counts[None, :].astype(jnp.float32)
    logits = jnp.log(class_w + _EPS)
    logits_pp = logits[labels]
    g = jax.random.gumbel(kc, (_N, _NUM_NEG, _NUM_LABELS), jnp.float32)
    neg_cls = jnp.argmax(g + logits_pp[:, None, :], axis=-1).astype(jnp.int32)
    u2 = jax.random.uniform(ku, (_N, _NUM_NEG))

    counts_p = jnp.concatenate(
        [counts.astype(jnp.int32), jnp.ones((_LT - _NUM_LABELS,), jnp.int32)])
    offsets_p = jnp.concatenate(
        [offsets.astype(jnp.int32), jnp.zeros((_LT - _NUM_LABELS,), jnp.int32)])

    sorted_feats = _permute(features, order)
    pos_loss, neg_loss = _main(
        sorted_feats, features, labels,
        u.T, u2.T, neg_cls.T,
        counts_p, offsets_p)
    loss = pos_loss.mean() + neg_loss.mean() * _NEG_W
    return (loss, pos_loss, neg_loss)
